# Initial kernel scaffold; baseline (speedup 1.0000x reference)
import jax
import jax.numpy as jnp
from jax.experimental import pallas as pl


def kernel(index, length, data):
    raise NotImplementedError("write your pallas kernel here")


# speedup vs baseline: 13.6214x; 13.6214x over previous
import functools

import jax
import jax.numpy as jnp
from jax import lax
from jax.experimental import pallas as pl
from jax.experimental.pallas import tpu as pltpu
from jax.experimental.pallas import tpu_sc as plsc

CYCLE = 168
LEN = 336
CHAN = 128
BATCH = 1024

_NC = 2
_NS = 16
_NW = _NC * _NS
_BPW = BATCH // _NW

_mesh = plsc.VectorSubcoreMesh(core_axis_name="c", subcore_axis_name="s")


@functools.partial(
    pl.kernel,
    out_type=jax.ShapeDtypeStruct((BATCH, LEN, CHAN), jnp.float32),
    mesh=_mesh,
    scratch_types=[
        pltpu.VMEM((3 * CYCLE, CHAN), jnp.float32),
        pltpu.VMEM((_BPW,), jnp.int32),
    ],
)
def _recurrent_sc(idx_hbm, data_hbm, out_hbm, ddd_v, idx_v):
    c = lax.axis_index("c")
    s = lax.axis_index("s")
    wid = s * _NC + c
    base = wid * _BPW

    pltpu.sync_copy(data_hbm, ddd_v.at[pl.ds(0, CYCLE)])
    pltpu.sync_copy(data_hbm, ddd_v.at[pl.ds(CYCLE, CYCLE)])
    pltpu.sync_copy(data_hbm, ddd_v.at[pl.ds(2 * CYCLE, CYCLE)])
    pltpu.sync_copy(idx_hbm.at[pl.ds(base, _BPW)], idx_v)

    for g in range(_BPW // 16):
        vec = idx_v[pl.ds(g * 16, 16)]
        for lane in range(16):
            start = vec[lane]
            b = base + g * 16 + lane
            pltpu.sync_copy(ddd_v.at[pl.ds(start, LEN)], out_hbm.at[b])


def kernel(index, length, data):
    shift = jnp.asarray(length, jnp.int32) - LEN
    eff = ((index.astype(jnp.int32) + shift) % CYCLE).astype(jnp.int32)
    return _recurrent_sc(eff, data)


# speedup vs baseline: 13.6700x; 1.0036x over previous
import functools

import jax
import jax.numpy as jnp
from jax import lax
from jax.experimental import pallas as pl
from jax.experimental.pallas import tpu as pltpu
from jax.experimental.pallas import tpu_sc as plsc

CYCLE = 168
LEN = 336
CHAN = 128
BATCH = 1024

_NC = 2
_NS = 16
_NW = _NC * _NS
_BPW = BATCH // _NW

_mesh = plsc.VectorSubcoreMesh(core_axis_name="c", subcore_axis_name="s")


@functools.partial(
    pl.kernel,
    out_type=jax.ShapeDtypeStruct((BATCH, LEN, CHAN), jnp.float32),
    mesh=_mesh,
    scratch_types=[
        pltpu.VMEM((3 * CYCLE, CHAN), jnp.float32),
        pltpu.VMEM((_BPW,), jnp.int32),
        pltpu.SemaphoreType.DMA,
    ],
)
def _recurrent_sc(idx_hbm, data_hbm, out_hbm, ddd_v, idx_v, sem):
    c = lax.axis_index("c")
    s = lax.axis_index("s")
    wid = s * _NC + c
    base = wid * _BPW

    pltpu.sync_copy(data_hbm, ddd_v.at[pl.ds(0, CYCLE)])
    pltpu.sync_copy(data_hbm, ddd_v.at[pl.ds(CYCLE, CYCLE)])
    pltpu.sync_copy(data_hbm, ddd_v.at[pl.ds(2 * CYCLE, CYCLE)])
    pltpu.sync_copy(idx_hbm.at[pl.ds(base, _BPW)], idx_v)

    handles = []
    for g in range(_BPW // 16):
        vec = idx_v[pl.ds(g * 16, 16)]
        for lane in range(16):
            start = vec[lane]
            b = base + g * 16 + lane
            handles.append(
                pltpu.async_copy(ddd_v.at[pl.ds(start, LEN)], out_hbm.at[b], sem)
            )
    for h in handles:
        h.wait()


def kernel(index, length, data):
    shift = jnp.asarray(length, jnp.int32) - LEN
    eff = ((index.astype(jnp.int32) + shift) % CYCLE).astype(jnp.int32)
    return _recurrent_sc(eff, data)


# speedup vs baseline: 13.8891x; 1.0160x over previous
import functools

import jax
import jax.numpy as jnp
from jax import lax
from jax.experimental import pallas as pl
from jax.experimental.pallas import tpu as pltpu
from jax.experimental.pallas import tpu_sc as plsc

CYCLE = 168
LEN = 336
CHAN = 128
BATCH = 1024

_NC = 2
_NS = 16
_NW = _NC * _NS
_BPW = BATCH // _NW

_mesh = plsc.VectorSubcoreMesh(core_axis_name="c", subcore_axis_name="s")


@functools.partial(
    pl.kernel,
    out_type=jax.ShapeDtypeStruct((BATCH, LEN, CHAN), jnp.float32),
    mesh=_mesh,
    scratch_types=[
        pltpu.VMEM((3 * CYCLE, CHAN), jnp.float32),
        pltpu.VMEM((_BPW,), jnp.int32),
        pltpu.VMEM_SHARED((3 * CYCLE, CHAN), jnp.float32),
        pltpu.SemaphoreType.DMA,
    ],
)
def _recurrent_sc(idx_hbm, data_hbm, out_hbm, ddd_v, idx_v, ddd_sh, sem):
    c = lax.axis_index("c")
    s = lax.axis_index("s")
    wid = s * _NC + c
    base = wid * _BPW

    @pl.when(s == 0)
    def _():
        pltpu.sync_copy(data_hbm, ddd_sh.at[pl.ds(0, CYCLE)])
        pltpu.sync_copy(data_hbm, ddd_sh.at[pl.ds(CYCLE, CYCLE)])
        pltpu.sync_copy(data_hbm, ddd_sh.at[pl.ds(2 * CYCLE, CYCLE)])

    pltpu.sync_copy(data_hbm, ddd_v.at[pl.ds(0, CYCLE)])
    pltpu.sync_copy(data_hbm, ddd_v.at[pl.ds(CYCLE, CYCLE)])
    pltpu.sync_copy(data_hbm, ddd_v.at[pl.ds(2 * CYCLE, CYCLE)])
    pltpu.sync_copy(idx_hbm.at[pl.ds(base, _BPW)], idx_v)
    plsc.subcore_barrier()

    handles = []
    for g in range(_BPW // 16):
        vec = idx_v[pl.ds(g * 16, 16)]
        for lane in range(16):
            start = vec[lane]
            b = base + g * 16 + lane
            src = ddd_v if (g * 16 + lane) % 2 == 0 else ddd_sh
            handles.append(
                pltpu.async_copy(src.at[pl.ds(start, LEN)], out_hbm.at[b], sem)
            )
    for h in handles:
        h.wait()


def kernel(index, length, data):
    shift = jnp.asarray(length, jnp.int32) - LEN
    eff = ((index.astype(jnp.int32) + shift) % CYCLE).astype(jnp.int32)
    return _recurrent_sc(eff, data)


# speedup vs baseline: 15.2407x; 1.0973x over previous
import functools

import jax
import jax.numpy as jnp
from jax.experimental import pallas as pl
from jax.experimental.pallas import tpu as pltpu

CYCLE = 168
LEN = 336
CHAN = 128
BATCH = 1024

_BB = 8


def _tc_body(idx_sm, ddd_ref, out_ref):
    pid = pl.program_id(0)
    for j in range(_BB):
        start = idx_sm[pid * _BB + j]
        out_ref[j] = ddd_ref[pl.ds(start, LEN), :]


@jax.jit
def _tc_call(eff, ddd):
    grid_spec = pltpu.PrefetchScalarGridSpec(
        num_scalar_prefetch=1,
        grid=(BATCH // _BB,),
        in_specs=[pl.BlockSpec((3 * CYCLE, CHAN), lambda i, *_: (0, 0))],
        out_specs=pl.BlockSpec((_BB, LEN, CHAN), lambda i, *_: (i, 0, 0)),
    )
    return pl.pallas_call(
        _tc_body,
        grid_spec=grid_spec,
        out_shape=jax.ShapeDtypeStruct((BATCH, LEN, CHAN), jnp.float32),
    )(eff, ddd)


def kernel(index, length, data):
    shift = jnp.asarray(length, jnp.int32) - LEN
    eff = ((index.astype(jnp.int32) + shift) % CYCLE).astype(jnp.int32)
    ddd = jnp.concatenate([data, data, data], axis=0)
    return _tc_call(eff, ddd)
